# Initial kernel scaffold; baseline (speedup 1.0000x reference)
import functools

import jax
import jax.numpy as jnp
from jax import lax
from jax.experimental import pallas as pl
from jax.experimental.pallas import tpu as pltpu
from jax.experimental.pallas import tpu_sc as plsc

_NC = 2
_NS = 16
_NW = _NC * _NS
_L = 16


def _sc_spmm_partials(indices, values, b):
    E = values.shape[0]
    N, D = b.shape
    EPW = E // _NW
    C = 80
    NCH = EPW // C
    RPT = N // _NS
    ZR = 125
    assert EPW * _NW == E and NCH * C == EPW and RPT * _NS == N
    assert RPT % ZR == 0 and D % _L == 0

    mesh = plsc.VectorSubcoreMesh(core_axis_name="c", subcore_axis_name="s")

    @functools.partial(
        pl.kernel,
        mesh=mesh,
        out_type=jax.ShapeDtypeStruct((_NC, N, D), jnp.float32),
        scratch_types=dict(
            acc_sh=pltpu.VMEM_SHARED((N, D), jnp.float32),
            src_v=pltpu.VMEM((EPW,), jnp.int32),
            vals_v=pltpu.VMEM((EPW,), jnp.float32),
            dst_v=pltpu.VMEM((C,), jnp.int32),
            rows_v=pltpu.VMEM((C, D), jnp.float32),
            zero_v=pltpu.VMEM((ZR, D), jnp.float32),
            gsem=pltpu.SemaphoreType.DMA,
        ),
    )
    def spmm(idx_hbm, vals_hbm, b_hbm, out_hbm,
             acc_sh, src_v, vals_v, dst_v, rows_v, zero_v, gsem):
        cid = lax.axis_index("c")
        sid = lax.axis_index("s")
        w = sid * _NC + cid
        ebase = w * EPW

        zvec = jnp.zeros((_L,), jnp.float32)

        def zrow(r, _):
            for j in range(D // _L):
                zero_v[r, pl.ds(j * _L, _L)] = zvec
            return 0

        lax.fori_loop(0, ZR, zrow, 0)
        for z in range(RPT // ZR):
            pltpu.sync_copy(zero_v, acc_sh.at[pl.ds(sid * RPT + z * ZR, ZR)])

        pltpu.sync_copy(idx_hbm.at[1, pl.ds(ebase, EPW)], src_v)
        pltpu.sync_copy(vals_hbm.at[pl.ds(ebase, EPW)], vals_v)

        plsc.subcore_barrier()

        def chunk(i, _):
            cb = i * C
            pltpu.sync_copy(idx_hbm.at[0, pl.ds(ebase + cb, C)], dst_v)
            pltpu.async_copy(b_hbm.at[src_v.at[pl.ds(cb, C)]], rows_v,
                             gsem).wait()

            def scale(e, _):
                v16 = plsc.load_gather(
                    vals_v, [jnp.full((_L,), cb + e, jnp.int32)])
                for j in range(D // _L):
                    sl = pl.ds(j * _L, _L)
                    rows_v[e, sl] = rows_v[e, sl] * v16
                return 0

            lax.fori_loop(0, C, scale, 0, unroll=2)

            pltpu.sync_copy(rows_v, acc_sh.at[dst_v], add=True)
            return 0

        lax.fori_loop(0, NCH, chunk, 0)

        plsc.subcore_barrier()

        pltpu.sync_copy(acc_sh.at[pl.ds(sid * RPT, RPT)],
                        out_hbm.at[cid, pl.ds(sid * RPT, RPT)])

    return spmm(indices, values, b)


def _merge_body(p_ref, o_ref):
    o_ref[...] = p_ref[0] + p_ref[1]


def kernel(indices, values, shape, b):
    del shape
    N, D = b.shape
    partials = _sc_spmm_partials(indices, values, b)
    nblk = 8
    rb = N // nblk
    return pl.pallas_call(
        _merge_body,
        grid=(nblk,),
        in_specs=[pl.BlockSpec((_NC, rb, D), lambda i: (0, i, 0))],
        out_specs=pl.BlockSpec((rb, D), lambda i: (i, 0)),
        out_shape=jax.ShapeDtypeStruct((N, D), jnp.float32),
    )(partials)


# speedup vs baseline: 5.1324x; 5.1324x over previous
import functools

import jax
import jax.numpy as jnp
from jax import lax
from jax.experimental import pallas as pl
from jax.experimental.pallas import tpu as pltpu
from jax.experimental.pallas import tpu_sc as plsc

_NC = 2
_NS = 16
_NW = _NC * _NS
_L = 16


def _sc_spmm_partials(dst, src, values, b):
    E = values.shape[0]
    N, D = b.shape
    EPW = E // _NW
    C = 80
    NCH = EPW // C
    RT = 624
    ZR = 104
    TAIL = N - RT * _NS
    assert EPW * _NW == E and NCH * C == EPW
    assert RT % ZR == 0 and TAIL % 8 == 0 and TAIL <= ZR and D % _L == 0

    mesh = plsc.VectorSubcoreMesh(core_axis_name="c", subcore_axis_name="s")

    @functools.partial(
        pl.kernel,
        mesh=mesh,
        out_type=jax.ShapeDtypeStruct((_NC, N, D), jnp.float32),
        scratch_types=dict(
            acc_sh=pltpu.VMEM_SHARED((N, D), jnp.float32),
            vals_v=pltpu.VMEM((EPW,), jnp.float32),
            srcc_v=pltpu.VMEM((C,), jnp.int32),
            dst_v=pltpu.VMEM((C,), jnp.int32),
            rows_v=pltpu.VMEM((C, D), jnp.float32),
            zero_v=pltpu.VMEM((ZR, D), jnp.float32),
            gsem=pltpu.SemaphoreType.DMA,
        ),
    )
    def spmm(dst_hbm, src_hbm, vals_hbm, b_hbm, out_hbm,
             acc_sh, vals_v, srcc_v, dst_v, rows_v, zero_v, gsem):
        cid = lax.axis_index("c")
        sid = lax.axis_index("s")
        w = sid * _NC + cid
        ebase = w * EPW

        zvec = jnp.zeros((_L,), jnp.float32)

        def zrow(r, _):
            for j in range(D // _L):
                zero_v[r, pl.ds(j * _L, _L)] = zvec
            return 0

        lax.fori_loop(0, ZR, zrow, 0)
        for z in range(RT // ZR):
            off = pl.multiple_of(sid * RT + z * ZR, 8)
            pltpu.sync_copy(zero_v, acc_sh.at[pl.ds(off, ZR)])
        pltpu.sync_copy(zero_v.at[pl.ds(0, TAIL)],
                        acc_sh.at[pl.ds(N - TAIL, TAIL)])

        pltpu.sync_copy(vals_hbm.at[pl.ds(ebase, EPW)], vals_v)

        plsc.subcore_barrier()

        def chunk(i, _):
            cb = i * C
            pltpu.sync_copy(dst_hbm.at[pl.ds(ebase + cb, C)], dst_v)
            pltpu.sync_copy(src_hbm.at[pl.ds(ebase + cb, C)], srcc_v)
            pltpu.async_copy(b_hbm.at[srcc_v], rows_v, gsem).wait()

            dnums = lax.GatherDimensionNumbers(
                offset_dims=(), collapsed_slice_dims=(0,),
                start_index_map=(0,))

            def scale(g, _):
                vals16 = vals_v[pl.ds(cb + g * _L, _L)]
                for e in range(_L):
                    lane = jnp.full((_L, 1), e, jnp.int32)
                    v16 = lax.gather(
                        vals16, lane, dnums, (1,),
                        mode=lax.GatherScatterMode.PROMISE_IN_BOUNDS)
                    for j in range(D // _L):
                        sl = pl.ds(j * _L, _L)
                        rows_v[g * _L + e, sl] = rows_v[g * _L + e, sl] * v16
                return 0

            lax.fori_loop(0, C // _L, scale, 0)

            pltpu.sync_copy(rows_v, acc_sh.at[dst_v], add=True)
            return 0

        lax.fori_loop(0, NCH, chunk, 0)

        plsc.subcore_barrier()

        off = pl.multiple_of(sid * RT, 8)
        pltpu.sync_copy(acc_sh.at[pl.ds(off, RT)],
                        out_hbm.at[cid, pl.ds(off, RT)])
        pltpu.sync_copy(acc_sh.at[pl.ds(N - TAIL, TAIL)],
                        out_hbm.at[cid, pl.ds(N - TAIL, TAIL)])

    return spmm(dst, src, values, b)


def _merge_body(p_ref, o_ref):
    o_ref[...] = p_ref[0] + p_ref[1]


def kernel(indices, values, shape, b):
    del shape
    N, D = b.shape
    partials = _sc_spmm_partials(indices[0], indices[1], values, b)
    nblk = 10
    rb = N // nblk
    return pl.pallas_call(
        _merge_body,
        grid=(nblk,),
        in_specs=[pl.BlockSpec((_NC, rb, D), lambda i: (0, i, 0))],
        out_specs=pl.BlockSpec((rb, D), lambda i: (i, 0)),
        out_shape=jax.ShapeDtypeStruct((N, D), jnp.float32),
    )(partials)


# speedup vs baseline: 10.7348x; 2.0916x over previous
import functools

import jax
import jax.numpy as jnp
from jax import lax
from jax.experimental import pallas as pl
from jax.experimental.pallas import tpu as pltpu
from jax.experimental.pallas import tpu_sc as plsc

_NC = 2
_NS = 16
_NW = _NC * _NS
_L = 16


def _sc_spmm_partials(dst, src, values, b):
    E = values.shape[0]
    N, D = b.shape
    EPW = E // _NW
    C = 64
    NF = EPW // C
    REM = NF * C
    CR = EPW - REM
    RT = 624
    ZR = 104
    TAIL = N - RT * _NS
    assert EPW * _NW == E and NF % 2 == 0 and CR in (0, _L)
    assert RT % ZR == 0 and TAIL % 8 == 0 and TAIL <= ZR and D % _L == 0

    mesh = plsc.VectorSubcoreMesh(core_axis_name="c", subcore_axis_name="s")

    @functools.partial(
        pl.kernel,
        mesh=mesh,
        out_type=jax.ShapeDtypeStruct((_NC, N, D), jnp.float32),
        scratch_types=dict(
            acc_sh=pltpu.VMEM_SHARED((N, D), jnp.float32),
            vals_v=pltpu.VMEM((EPW,), jnp.float32),
            sidx_a=pltpu.VMEM((C,), jnp.int32),
            sidx_b=pltpu.VMEM((C,), jnp.int32),
            didx_a=pltpu.VMEM((C,), jnp.int32),
            didx_b=pltpu.VMEM((C,), jnp.int32),
            rems_v=pltpu.VMEM((_L,), jnp.int32),
            remd_v=pltpu.VMEM((_L,), jnp.int32),
            rows_a=pltpu.VMEM((C, D), jnp.float32),
            rows_b=pltpu.VMEM((C, D), jnp.float32),
            zero_v=pltpu.VMEM((ZR, D), jnp.float32),
            gsem_a=pltpu.SemaphoreType.DMA,
            gsem_b=pltpu.SemaphoreType.DMA,
            ssem_a=pltpu.SemaphoreType.DMA,
            ssem_b=pltpu.SemaphoreType.DMA,
            dsem_a=pltpu.SemaphoreType.DMA,
            dsem_b=pltpu.SemaphoreType.DMA,
        ),
    )
    def spmm(dst_hbm, src_hbm, vals_hbm, b_hbm, out_hbm,
             acc_sh, vals_v, sidx_a, sidx_b, didx_a, didx_b, rems_v, remd_v,
             rows_a, rows_b, zero_v, gsem_a, gsem_b, ssem_a, ssem_b,
             dsem_a, dsem_b):
        cid = lax.axis_index("c")
        sid = lax.axis_index("s")
        w = sid * _NC + cid
        ebase = w * EPW

        zvec = jnp.zeros((_L,), jnp.float32)

        def zrow(r, _):
            for j in range(D // _L):
                zero_v[r, pl.ds(j * _L, _L)] = zvec
            return 0

        lax.fori_loop(0, ZR, zrow, 0)
        for z in range(RT // ZR):
            off = pl.multiple_of(sid * RT + z * ZR, 8)
            pltpu.sync_copy(zero_v, acc_sh.at[pl.ds(off, ZR)])
        pltpu.sync_copy(zero_v.at[pl.ds(0, TAIL)],
                        acc_sh.at[pl.ds(N - TAIL, TAIL)])

        pltpu.sync_copy(vals_hbm.at[pl.ds(ebase, EPW)], vals_v)

        plsc.subcore_barrier()

        def sidx(i, buf, sem):
            return pltpu.make_async_copy(
                src_hbm.at[pl.ds(ebase + i * C, C)], buf, sem)

        def didx(i, buf, sem):
            return pltpu.make_async_copy(
                dst_hbm.at[pl.ds(ebase + i * C, C)], buf, sem)

        def gat(buf_idx, buf, sem):
            return pltpu.make_async_copy(b_hbm.at[buf_idx], buf, sem)

        dnums = lax.GatherDimensionNumbers(
            offset_dims=(), collapsed_slice_dims=(0,), start_index_map=(0,))

        def scale(buf, cb, ngroups):
            def group(g, _):
                vals16 = vals_v[pl.ds(cb + g * _L, _L)]
                for e in range(_L):
                    lane = jnp.full((_L, 1), e, jnp.int32)
                    v16 = lax.gather(
                        vals16, lane, dnums, (1,),
                        mode=lax.GatherScatterMode.PROMISE_IN_BOUNDS)
                    for j in range(D // _L):
                        sl = pl.ds(j * _L, _L)
                        buf[g * _L + e, sl] = buf[g * _L + e, sl] * v16
                return 0

            lax.fori_loop(0, ngroups, group, 0)

        def scat(buf, buf_idx):
            pltpu.sync_copy(buf, acc_sh.at[buf_idx], add=True)

        sidx(0, sidx_a, ssem_a).start()
        sidx(1, sidx_b, ssem_b).start()
        didx(0, didx_a, dsem_a).start()
        sidx(0, sidx_a, ssem_a).wait()
        gat(sidx_a, rows_a, gsem_a).start()
        sidx(1, sidx_b, ssem_b).wait()
        gat(sidx_b, rows_b, gsem_b).start()

        def phase(i, s_buf, s_sem, d_buf, d_sem, r_buf, g_sem,
                  d_next, dsem_next, prefetch):
            if prefetch:
                didx(i + 1, d_next, dsem_next).start()
            gat(s_buf, r_buf, g_sem).wait()
            if prefetch:
                sidx(i + 2, s_buf, s_sem).start()
            scale(r_buf, i * C, C // _L)
            didx(i, d_buf, d_sem).wait()
            scat(r_buf, d_buf)
            if prefetch:
                sidx(i + 2, s_buf, s_sem).wait()
                gat(s_buf, r_buf, g_sem).start()

        def pair(s, _):
            i0 = 2 * s
            phase(i0, sidx_a, ssem_a, didx_a, dsem_a, rows_a, gsem_a,
                  didx_b, dsem_b, True)
            phase(i0 + 1, sidx_b, ssem_b, didx_b, dsem_b, rows_b, gsem_b,
                  didx_a, dsem_a, True)
            return 0

        lax.fori_loop(0, NF // 2 - 1, pair, 0)

        i0 = NF - 2
        didx(i0 + 1, didx_b, dsem_b).start()
        gat(sidx_a, rows_a, gsem_a).wait()
        if CR:
            pltpu.make_async_copy(
                src_hbm.at[pl.ds(ebase + REM, CR)], rems_v, ssem_a).start()
        scale(rows_a, i0 * C, C // _L)
        didx(i0, didx_a, dsem_a).wait()
        scat(rows_a, didx_a)
        if CR:
            pltpu.make_async_copy(
                dst_hbm.at[pl.ds(ebase + REM, CR)], remd_v, dsem_a).start()
            pltpu.make_async_copy(
                src_hbm.at[pl.ds(ebase + REM, CR)], rems_v, ssem_a).wait()
            pltpu.make_async_copy(
                b_hbm.at[rems_v], rows_a.at[pl.ds(0, CR)], gsem_a).start()
        gat(sidx_b, rows_b, gsem_b).wait()
        scale(rows_b, (NF - 1) * C, C // _L)
        didx(NF - 1, didx_b, dsem_b).wait()
        scat(rows_b, didx_b)
        if CR:
            pltpu.make_async_copy(
                b_hbm.at[rems_v], rows_a.at[pl.ds(0, CR)], gsem_a).wait()
            scale(rows_a, REM, CR // _L)
            pltpu.make_async_copy(
                dst_hbm.at[pl.ds(ebase + REM, CR)], remd_v, dsem_a).wait()
            pltpu.sync_copy(rows_a.at[pl.ds(0, CR)], acc_sh.at[remd_v],
                            add=True)

        plsc.subcore_barrier()

        off = pl.multiple_of(sid * RT, 8)
        pltpu.sync_copy(acc_sh.at[pl.ds(off, RT)],
                        out_hbm.at[cid, pl.ds(off, RT)])
        pltpu.sync_copy(acc_sh.at[pl.ds(N - TAIL, TAIL)],
                        out_hbm.at[cid, pl.ds(N - TAIL, TAIL)])

    return spmm(dst, src, values, b)


def _merge_body(p_ref, o_ref):
    o_ref[...] = p_ref[0] + p_ref[1]


def kernel(indices, values, shape, b):
    del shape
    N, D = b.shape
    partials = _sc_spmm_partials(indices[0], indices[1], values, b)
    nblk = 10
    rb = N // nblk
    return pl.pallas_call(
        _merge_body,
        grid=(nblk,),
        in_specs=[pl.BlockSpec((_NC, rb, D), lambda i: (0, i, 0))],
        out_specs=pl.BlockSpec((rb, D), lambda i: (i, 0)),
        out_shape=jax.ShapeDtypeStruct((N, D), jnp.float32),
    )(partials)


# speedup vs baseline: 11.9836x; 1.1163x over previous
import functools

import jax
import jax.numpy as jnp
from jax import lax
from jax.experimental import pallas as pl
from jax.experimental.pallas import tpu as pltpu
from jax.experimental.pallas import tpu_sc as plsc

_NC = 2
_NS = 16
_NW = _NC * _NS
_L = 16


def _sc_spmm_partials(dst, src, values, b):
    E = values.shape[0]
    N, D = b.shape
    EPW = E // _NW
    C = 64
    NF = EPW // C
    REM = NF * C
    CR = EPW - REM
    RT = 624
    ZR = 48
    TAIL = N - RT * _NS
    assert EPW * _NW == E and NF % 4 == 0 and NF >= 12 and CR in (0, _L)
    assert RT % ZR == 0 and TAIL % 8 == 0 and TAIL <= ZR and D % _L == 0

    mesh = plsc.VectorSubcoreMesh(core_axis_name="c", subcore_axis_name="s")

    scratch = dict(
        acc_sh=pltpu.VMEM_SHARED((N, D), jnp.float32),
        vals_v=pltpu.VMEM((EPW,), jnp.float32),
        zero_v=pltpu.VMEM((ZR, D), jnp.float32),
        rems_v=pltpu.VMEM((_L,), jnp.int32),
        remd_v=pltpu.VMEM((_L,), jnp.int32),
    )
    for k in range(4):
        scratch[f"rows{k}"] = pltpu.VMEM((C, D), jnp.float32)
        scratch[f"sb{k}"] = pltpu.VMEM((C,), jnp.int32)
        scratch[f"db{k}"] = pltpu.VMEM((C,), jnp.int32)
        scratch[f"gsem{k}"] = pltpu.SemaphoreType.DMA
        scratch[f"ssem{k}"] = pltpu.SemaphoreType.DMA
        scratch[f"dsem{k}"] = pltpu.SemaphoreType.DMA
        scratch[f"csem{k}"] = pltpu.SemaphoreType.DMA

    @functools.partial(
        pl.kernel,
        mesh=mesh,
        out_type=jax.ShapeDtypeStruct((_NC, N, D), jnp.float32),
        scratch_types=scratch,
    )
    def spmm(dst_hbm, src_hbm, vals_hbm, b_hbm, out_hbm, **scr):
        acc_sh = scr["acc_sh"]
        vals_v = scr["vals_v"]
        zero_v = scr["zero_v"]
        rems_v = scr["rems_v"]
        remd_v = scr["remd_v"]
        rows = [scr[f"rows{k}"] for k in range(4)]
        sb = [scr[f"sb{k}"] for k in range(4)]
        db = [scr[f"db{k}"] for k in range(4)]
        gsem = [scr[f"gsem{k}"] for k in range(4)]
        ssem = [scr[f"ssem{k}"] for k in range(4)]
        dsem = [scr[f"dsem{k}"] for k in range(4)]
        csem = [scr[f"csem{k}"] for k in range(4)]

        cid = lax.axis_index("c")
        sid = lax.axis_index("s")
        w = sid * _NC + cid
        ebase = w * EPW

        zvec = jnp.zeros((_L,), jnp.float32)

        def zrow(r, _):
            for j in range(D // _L):
                zero_v[r, pl.ds(j * _L, _L)] = zvec
            return 0

        lax.fori_loop(0, ZR, zrow, 0)
        for z in range(RT // ZR):
            off = pl.multiple_of(sid * RT + z * ZR, 8)
            pltpu.sync_copy(zero_v, acc_sh.at[pl.ds(off, ZR)])
        pltpu.sync_copy(zero_v.at[pl.ds(0, TAIL)],
                        acc_sh.at[pl.ds(N - TAIL, TAIL)])

        pltpu.sync_copy(vals_hbm.at[pl.ds(ebase, EPW)], vals_v)

        plsc.subcore_barrier()

        def sidx(i, k):
            return pltpu.make_async_copy(
                src_hbm.at[pl.ds(ebase + i * C, C)], sb[k], ssem[k])

        def didx(i, k):
            return pltpu.make_async_copy(
                dst_hbm.at[pl.ds(ebase + i * C, C)], db[k], dsem[k])

        def gat(k):
            return pltpu.make_async_copy(b_hbm.at[sb[k]], rows[k], gsem[k])

        def scat(k):
            return pltpu.async_copy(rows[k], acc_sh.at[db[k]], csem[k],
                                    add=True)

        def scat_wait(k):
            pltpu.make_async_copy(rows[k], acc_sh.at[db[k]], csem[k]).wait()

        dnums = lax.GatherDimensionNumbers(
            offset_dims=(), collapsed_slice_dims=(0,), start_index_map=(0,))

        def scale(buf, cb, ngroups):
            def group(g, _):
                vals16 = vals_v[pl.ds(cb + g * _L, _L)]
                for e in range(_L):
                    lane = jnp.full((_L, 1), e, jnp.int32)
                    v16 = lax.gather(
                        vals16, lane, dnums, (1,),
                        mode=lax.GatherScatterMode.PROMISE_IN_BOUNDS)
                    for j in range(D // _L):
                        sl = pl.ds(j * _L, _L)
                        buf[g * _L + e, sl] = buf[g * _L + e, sl] * v16
                return 0

            lax.fori_loop(0, ngroups, group, 0)

        def phase(i, k, fire_sidx=True, fire_didx=True, fire_gat=True,
                  wait_scat=True):
            pltpu.make_async_copy(b_hbm.at[sb[k]], rows[k], gsem[k]).wait()
            if fire_sidx:
                sidx(i + 3, (k + 3) % 4).start()
            if wait_scat:
                scat_wait((k + 2) % 4)
            if fire_didx:
                didx(i + 2, (k + 2) % 4).start()
            scale(rows[k], i * C, C // _L)
            didx(i, k).wait()
            scat(k)
            if fire_gat:
                sidx(i + 2, (k + 2) % 4).wait()
                gat((k + 2) % 4).start()

        sidx(0, 0).start()
        sidx(1, 1).start()
        sidx(2, 2).start()
        didx(0, 0).start()
        didx(1, 1).start()
        sidx(0, 0).wait()
        gat(0).start()
        sidx(1, 1).wait()
        gat(1).start()

        phase(0, 0, wait_scat=False)
        phase(1, 1, wait_scat=False)
        phase(2, 2)
        phase(3, 3)

        def quad(g, _):
            i0 = 4 * g + 4
            phase(i0, 0)
            phase(i0 + 1, 1)
            phase(i0 + 2, 2)
            phase(i0 + 3, 3)
            return 0

        lax.fori_loop(0, NF // 4 - 2, quad, 0)

        phase(NF - 4, 0, fire_sidx=True, fire_didx=True, fire_gat=True)
        phase(NF - 3, 1, fire_sidx=False, fire_didx=True, fire_gat=True)
        phase(NF - 2, 2, fire_sidx=False, fire_didx=False, fire_gat=False,
              wait_scat=False)
        if CR:
            pltpu.make_async_copy(
                src_hbm.at[pl.ds(ebase + REM, CR)], rems_v, ssem[0]).start()
            pltpu.make_async_copy(
                dst_hbm.at[pl.ds(ebase + REM, CR)], remd_v, dsem[0]).start()
        phase(NF - 1, 3, fire_sidx=False, fire_didx=False, fire_gat=False,
              wait_scat=False)
        if CR:
            scat_wait(0)
            pltpu.make_async_copy(
                src_hbm.at[pl.ds(ebase + REM, CR)], rems_v, ssem[0]).wait()
            pltpu.make_async_copy(
                b_hbm.at[rems_v], rows[0].at[pl.ds(0, CR)], gsem[0]).start()
            pltpu.make_async_copy(
                b_hbm.at[rems_v], rows[0].at[pl.ds(0, CR)], gsem[0]).wait()
            scale(rows[0], REM, CR // _L)
            pltpu.make_async_copy(
                dst_hbm.at[pl.ds(ebase + REM, CR)], remd_v, dsem[0]).wait()
            pltpu.sync_copy(rows[0].at[pl.ds(0, CR)], acc_sh.at[remd_v],
                            add=True)
        for k in range(4) if not CR else range(1, 4):
            scat_wait(k)

        plsc.subcore_barrier()

        off = pl.multiple_of(sid * RT, 8)
        pltpu.sync_copy(acc_sh.at[pl.ds(off, RT)],
                        out_hbm.at[cid, pl.ds(off, RT)])
        pltpu.sync_copy(acc_sh.at[pl.ds(N - TAIL, TAIL)],
                        out_hbm.at[cid, pl.ds(N - TAIL, TAIL)])

    return spmm(dst, src, values, b)


def _merge_body(p_ref, o_ref):
    o_ref[...] = p_ref[0] + p_ref[1]


def kernel(indices, values, shape, b):
    del shape
    N, D = b.shape
    partials = _sc_spmm_partials(indices[0], indices[1], values, b)
    nblk = 10
    rb = N // nblk
    return pl.pallas_call(
        _merge_body,
        grid=(nblk,),
        in_specs=[pl.BlockSpec((_NC, rb, D), lambda i: (0, i, 0))],
        out_specs=pl.BlockSpec((rb, D), lambda i: (i, 0)),
        out_shape=jax.ShapeDtypeStruct((N, D), jnp.float32),
    )(partials)


# speedup vs baseline: 12.1585x; 1.0146x over previous
import functools

import jax
import jax.numpy as jnp
from jax import lax
from jax.experimental import pallas as pl
from jax.experimental.pallas import tpu as pltpu
from jax.experimental.pallas import tpu_sc as plsc

_NC = 2
_NS = 16
_NW = _NC * _NS
_L = 16


def _sc_spmm_partials(dst, src, values, b):
    E = values.shape[0]
    N, D = b.shape
    EPW = E // _NW
    C = 64
    NF = EPW // C
    REM = NF * C
    CR = EPW - REM
    RT = 624
    ZR = 48
    TAIL = N - RT * _NS
    assert EPW * _NW == E and NF % 4 == 0 and NF >= 12 and CR in (0, _L)
    assert RT % ZR == 0 and TAIL % 8 == 0 and TAIL <= ZR and D % _L == 0

    mesh = plsc.VectorSubcoreMesh(core_axis_name="c", subcore_axis_name="s")

    scratch = dict(
        acc_sh=pltpu.VMEM_SHARED((N, D), jnp.float32),
        vals_v=pltpu.VMEM((EPW,), jnp.float32),
        zero_v=pltpu.VMEM((ZR, D), jnp.float32),
        rems_v=pltpu.VMEM((_L,), jnp.int32),
        remd_v=pltpu.VMEM((_L,), jnp.int32),
    )
    for k in range(4):
        scratch[f"rows{k}"] = pltpu.VMEM((C, D), jnp.float32)
        scratch[f"sb{k}"] = pltpu.VMEM((C,), jnp.int32)
        scratch[f"db{k}"] = pltpu.VMEM((C,), jnp.int32)
        scratch[f"gsem{k}"] = pltpu.SemaphoreType.DMA
        scratch[f"ssem{k}"] = pltpu.SemaphoreType.DMA
        scratch[f"dsem{k}"] = pltpu.SemaphoreType.DMA
        scratch[f"csem{k}"] = pltpu.SemaphoreType.DMA

    @functools.partial(
        pl.kernel,
        mesh=mesh,
        out_type=jax.ShapeDtypeStruct((_NC, N, D), jnp.float32),
        scratch_types=scratch,
    )
    def spmm(dst_hbm, src_hbm, vals_hbm, b_hbm, out_hbm, **scr):
        acc_sh = scr["acc_sh"]
        vals_v = scr["vals_v"]
        zero_v = scr["zero_v"]
        rems_v = scr["rems_v"]
        remd_v = scr["remd_v"]
        rows = [scr[f"rows{k}"] for k in range(4)]
        sb = [scr[f"sb{k}"] for k in range(4)]
        db = [scr[f"db{k}"] for k in range(4)]
        gsem = [scr[f"gsem{k}"] for k in range(4)]
        ssem = [scr[f"ssem{k}"] for k in range(4)]
        dsem = [scr[f"dsem{k}"] for k in range(4)]
        csem = [scr[f"csem{k}"] for k in range(4)]

        cid = lax.axis_index("c")
        sid = lax.axis_index("s")
        w = sid * _NC + cid
        ebase = w * EPW

        zvec = jnp.zeros((_L,), jnp.float32)

        def zrow(r, _):
            for j in range(D // _L):
                zero_v[r, pl.ds(j * _L, _L)] = zvec
            return 0

        lax.fori_loop(0, ZR, zrow, 0)
        pltpu.make_async_copy(vals_hbm.at[pl.ds(ebase, EPW)], vals_v,
                              gsem[0]).start()
        zcps = []
        for z in range(RT // ZR):
            off = pl.multiple_of(sid * RT + z * ZR, 8)
            zcps.append(pltpu.make_async_copy(
                zero_v, acc_sh.at[pl.ds(off, ZR)], csem[0]))
        zcps.append(pltpu.make_async_copy(
            zero_v.at[pl.ds(0, TAIL)],
            acc_sh.at[pl.ds(N - TAIL, TAIL)], csem[0]))
        for cp in zcps:
            cp.start()
        for cp in zcps:
            cp.wait()
        pltpu.make_async_copy(vals_hbm.at[pl.ds(ebase, EPW)], vals_v,
                              gsem[0]).wait()

        plsc.subcore_barrier()

        def sidx(i, k):
            return pltpu.make_async_copy(
                src_hbm.at[pl.ds(ebase + i * C, C)], sb[k], ssem[k])

        def didx(i, k):
            return pltpu.make_async_copy(
                dst_hbm.at[pl.ds(ebase + i * C, C)], db[k], dsem[k])

        def gat(k):
            return pltpu.make_async_copy(b_hbm.at[sb[k]], rows[k], gsem[k])

        def scat(k):
            return pltpu.async_copy(rows[k], acc_sh.at[db[k]], csem[k],
                                    add=True)

        def scat_wait(k):
            pltpu.make_async_copy(rows[k], acc_sh.at[db[k]], csem[k]).wait()

        dnums = lax.GatherDimensionNumbers(
            offset_dims=(), collapsed_slice_dims=(0,), start_index_map=(0,))

        def scale(buf, cb, ngroups):
            def group(g, _):
                vals16 = vals_v[pl.ds(cb + g * _L, _L)]
                for e in range(_L):
                    lane = jnp.full((_L, 1), e, jnp.int32)
                    v16 = lax.gather(
                        vals16, lane, dnums, (1,),
                        mode=lax.GatherScatterMode.PROMISE_IN_BOUNDS)
                    for j in range(D // _L):
                        sl = pl.ds(j * _L, _L)
                        buf[g * _L + e, sl] = buf[g * _L + e, sl] * v16
                return 0

            lax.fori_loop(0, ngroups, group, 0)

        def phase(i, k, fire_sidx=True, fire_didx=True, fire_gat=True,
                  wait_scat=True):
            pltpu.make_async_copy(b_hbm.at[sb[k]], rows[k], gsem[k]).wait()
            if fire_sidx:
                sidx(i + 3, (k + 3) % 4).start()
            if wait_scat:
                scat_wait((k + 2) % 4)
            if fire_didx:
                didx(i + 2, (k + 2) % 4).start()
            scale(rows[k], i * C, C // _L)
            didx(i, k).wait()
            scat(k)
            if fire_gat:
                sidx(i + 2, (k + 2) % 4).wait()
                gat((k + 2) % 4).start()

        sidx(0, 0).start()
        sidx(1, 1).start()
        sidx(2, 2).start()
        didx(0, 0).start()
        didx(1, 1).start()
        sidx(0, 0).wait()
        gat(0).start()
        sidx(1, 1).wait()
        gat(1).start()

        phase(0, 0, wait_scat=False)
        phase(1, 1, wait_scat=False)
        phase(2, 2)
        phase(3, 3)

        def quad(g, _):
            i0 = 4 * g + 4
            phase(i0, 0)
            phase(i0 + 1, 1)
            phase(i0 + 2, 2)
            phase(i0 + 3, 3)
            return 0

        lax.fori_loop(0, NF // 4 - 2, quad, 0)

        phase(NF - 4, 0, fire_sidx=True, fire_didx=True, fire_gat=True)
        phase(NF - 3, 1, fire_sidx=False, fire_didx=True, fire_gat=True)
        phase(NF - 2, 2, fire_sidx=False, fire_didx=False, fire_gat=False,
              wait_scat=False)
        if CR:
            pltpu.make_async_copy(
                src_hbm.at[pl.ds(ebase + REM, CR)], rems_v, ssem[0]).start()
            pltpu.make_async_copy(
                dst_hbm.at[pl.ds(ebase + REM, CR)], remd_v, dsem[0]).start()
        phase(NF - 1, 3, fire_sidx=False, fire_didx=False, fire_gat=False,
              wait_scat=False)
        if CR:
            scat_wait(0)
            pltpu.make_async_copy(
                src_hbm.at[pl.ds(ebase + REM, CR)], rems_v, ssem[0]).wait()
            pltpu.make_async_copy(
                b_hbm.at[rems_v], rows[0].at[pl.ds(0, CR)], gsem[0]).start()
            pltpu.make_async_copy(
                b_hbm.at[rems_v], rows[0].at[pl.ds(0, CR)], gsem[0]).wait()
            scale(rows[0], REM, CR // _L)
            pltpu.make_async_copy(
                dst_hbm.at[pl.ds(ebase + REM, CR)], remd_v, dsem[0]).wait()
            pltpu.sync_copy(rows[0].at[pl.ds(0, CR)], acc_sh.at[remd_v],
                            add=True)
        for k in range(4) if not CR else range(1, 4):
            scat_wait(k)

        plsc.subcore_barrier()

        off = pl.multiple_of(sid * RT, 8)
        out1 = pltpu.make_async_copy(acc_sh.at[pl.ds(off, RT)],
                                     out_hbm.at[cid, pl.ds(off, RT)],
                                     gsem[0])
        out2 = pltpu.make_async_copy(acc_sh.at[pl.ds(N - TAIL, TAIL)],
                                     out_hbm.at[cid, pl.ds(N - TAIL, TAIL)],
                                     gsem[1])
        out1.start()
        out2.start()
        out1.wait()
        out2.wait()

    return spmm(dst, src, values, b)


def _merge_body(p_ref, o_ref):
    o_ref[...] = p_ref[0] + p_ref[1]


def kernel(indices, values, shape, b):
    del shape
    N, D = b.shape
    partials = _sc_spmm_partials(indices[0], indices[1], values, b)
    nblk = 10
    rb = N // nblk
    return pl.pallas_call(
        _merge_body,
        grid=(nblk,),
        in_specs=[pl.BlockSpec((_NC, rb, D), lambda i: (0, i, 0))],
        out_specs=pl.BlockSpec((rb, D), lambda i: (i, 0)),
        out_shape=jax.ShapeDtypeStruct((N, D), jnp.float32),
    )(partials)
